# baseline (speedup 1.0000x reference)
import functools

import jax
import jax.numpy as jnp
from jax import lax
from jax.experimental import pallas as pl
from jax.experimental.pallas import tpu as pltpu
from jax.experimental.pallas import tpu_sc as plsc

N = 10000
NIN = 512
NOUT = 512
FV = 13
_H = NOUT // 2

_QBITS = 12
_QSCALE = 512.0
_QOFF = 2048
_QMAX = 4095

_NC = 2
_NS = 16
_LANES = 16

NPAD = 10240
_CHUNK = 8
_FVA = 5
_FVB = FV - _FVA
_IPCA = _CHUNK * _FVA
_IPCB = _CHUNK * _FVB
_CH0 = 54
_CH1 = 26
_CHMAX = max(_CH0, _CH1)


def _quant(y):
    t = y * _QSCALE + (_QOFF + 0.5)
    t = jnp.clip(t, 0.0, float(_QMAX))
    return t.astype(jnp.int32)


def _mm_body(f_ref, w_ref, t_ref):
    y = jnp.dot(f_ref[...], w_ref[0], preferred_element_type=jnp.float32)
    qa = _quant(y[:, :_H])
    qb = _quant(y[:, _H:])
    t_ref[0] = qa | lax.shift_left(qb, 16)


def _matmul_offsets(features, w):
    fv = w.shape[0]
    blk = 2000
    grid = (N // blk, fv)
    return pl.pallas_call(
        _mm_body,
        grid=grid,
        in_specs=[
            pl.BlockSpec((blk, NIN), lambda nb, f: (nb, 0)),
            pl.BlockSpec((1, NIN, NOUT), lambda nb, f: (f, 0, 0)),
        ],
        out_specs=pl.BlockSpec((1, blk, _H), lambda nb, f: (f, nb, 0)),
        out_shape=jax.ShapeDtypeStruct((fv, N, _H), jnp.int32),
        compiler_params=pltpu.CompilerParams(
            dimension_semantics=("parallel", "parallel"),
        ),
    )(features, w)


_FVOFF = FV * _QOFF
_INV = 1.0 / _QSCALE


def _base_row(c, s):
    return jnp.where(
        c == 0,
        s * (_CH0 * _CHUNK),
        _NS * _CH0 * _CHUNK + s * (_CH1 * _CHUNK),
    )


def _sc_partial_body(t_hbm, ids_hbm, p_hbm, ids_v, g0, g1, outbuf, s0, s1):
    c = lax.axis_index("c")
    s = lax.axis_index("s")

    def issue(ch, buf, sem):
        pltpu.async_copy(
            t_hbm.at[ids_v.at[pl.ds(ch * _IPCA, _IPCA)]], buf, sem
        )

    def wait(buf, sem):
        pltpu.make_async_copy(
            t_hbm.at[ids_v.at[pl.ds(0, _IPCA)]], buf, sem
        ).wait()

    def consume(base_row, ch, buf):
        def grp(g, carry):
            sl = pl.ds(g * _LANES, _LANES)
            for r in range(_CHUNK):
                acc = buf[r * _FVA, sl]
                for f in range(1, _FVA):
                    acc = acc + buf[r * _FVA + f, sl]
                outbuf[r, sl] = acc
            return carry

        lax.fori_loop(0, _H // _LANES, grp, 0)
        pltpu.sync_copy(
            outbuf, p_hbm.at[pl.ds(base_row + ch * _CHUNK, _CHUNK)]
        )

    def run(base_row, n_chunks):
        pltpu.sync_copy(
            ids_hbm.at[pl.ds(base_row * _FVA, n_chunks * _IPCA)],
            ids_v.at[pl.ds(0, n_chunks * _IPCA)],
        )
        issue(0, g0, s0)
        issue(1, g1, s1)

        def steady(i, carry):
            ch = 2 * i
            wait(g0, s0)
            consume(base_row, ch, g0)
            issue(ch + 2, g0, s0)
            wait(g1, s1)
            consume(base_row, ch + 1, g1)
            issue(ch + 3, g1, s1)
            return carry

        lax.fori_loop(0, n_chunks // 2 - 1, steady, 0)
        wait(g0, s0)
        consume(base_row, n_chunks - 2, g0)
        wait(g1, s1)
        consume(base_row, n_chunks - 1, g1)

    @pl.when(c == 0)
    def _():
        run(s * (_CH0 * _CHUNK), _CH0)

    @pl.when(c == 1)
    def _():
        run(_NS * _CH0 * _CHUNK + s * (_CH1 * _CHUNK), _CH1)


def _sc_final_body(t_hbm, ids_hbm, p_hbm, bias_hbm, out_hbm, ids_v, bias_v,
                   g0, g1, p0, p1, outbuf, s0, s1, sp0, sp1):
    c = lax.axis_index("c")
    s = lax.axis_index("s")
    pltpu.sync_copy(bias_hbm, bias_v)

    def issue(base_row, ch, buf, sem, pbuf, psem):
        pltpu.async_copy(
            t_hbm.at[ids_v.at[pl.ds(ch * _IPCB, _IPCB)]], buf, sem
        )
        pltpu.async_copy(
            p_hbm.at[pl.ds(base_row + ch * _CHUNK, _CHUNK)], pbuf, psem
        )

    def wait(buf, sem, pbuf, psem):
        pltpu.make_async_copy(
            t_hbm.at[ids_v.at[pl.ds(0, _IPCB)]], buf, sem
        ).wait()
        pltpu.make_async_copy(
            p_hbm.at[pl.ds(0, _CHUNK)], pbuf, psem
        ).wait()

    def consume(base_row, ch, buf, pbuf):
        def grp(g, carry):
            sl = pl.ds(g * _LANES, _LANES)
            sh = pl.ds(_H + g * _LANES, _LANES)
            bl = bias_v[sl]
            bh = bias_v[sh]
            for r in range(_CHUNK):
                acc = pbuf[r, sl]
                for f in range(_FVB):
                    acc = acc + buf[r * _FVB + f, sl]
                lo = (acc & 0xFFFF) - _FVOFF
                hi = lax.shift_right_logical(acc, 16) - _FVOFF
                outbuf[r, sl] = lo.astype(jnp.float32) * _INV + bl
                outbuf[r, sh] = hi.astype(jnp.float32) * _INV + bh
            return carry

        lax.fori_loop(0, _H // _LANES, grp, 0)

        @pl.when(base_row + ch * _CHUNK < N)
        def _():
            pltpu.sync_copy(
                outbuf, out_hbm.at[pl.ds(base_row + ch * _CHUNK, _CHUNK)]
            )

    def run(base_row, n_chunks):
        pltpu.sync_copy(
            ids_hbm.at[pl.ds(base_row * _FVB, n_chunks * _IPCB)],
            ids_v.at[pl.ds(0, n_chunks * _IPCB)],
        )
        issue(base_row, 0, g0, s0, p0, sp0)
        issue(base_row, 1, g1, s1, p1, sp1)

        def steady(i, carry):
            ch = 2 * i
            wait(g0, s0, p0, sp0)
            consume(base_row, ch, g0, p0)
            issue(base_row, ch + 2, g0, s0, p0, sp0)
            wait(g1, s1, p1, sp1)
            consume(base_row, ch + 1, g1, p1)
            issue(base_row, ch + 3, g1, s1, p1, sp1)
            return carry

        lax.fori_loop(0, n_chunks // 2 - 1, steady, 0)
        wait(g0, s0, p0, sp0)
        consume(base_row, n_chunks - 2, g0, p0)
        wait(g1, s1, p1, sp1)
        consume(base_row, n_chunks - 1, g1, p1)

    @pl.when(c == 0)
    def _():
        run(s * (_CH0 * _CHUNK), _CH0)

    @pl.when(c == 1)
    def _():
        run(_NS * _CH0 * _CHUNK + s * (_CH1 * _CHUNK), _CH1)


def _sc_partial(t2a, ids_a):
    mesh = plsc.VectorSubcoreMesh(core_axis_name="c", subcore_axis_name="s")
    k = functools.partial(
        pl.kernel,
        out_type=jax.ShapeDtypeStruct((NPAD, _H), jnp.int32),
        mesh=mesh,
        scratch_types=[
            pltpu.VMEM((_CHMAX * _IPCA,), jnp.int32),
            pltpu.VMEM((_IPCA, _H), jnp.int32),
            pltpu.VMEM((_IPCA, _H), jnp.int32),
            pltpu.VMEM((_CHUNK, _H), jnp.int32),
            pltpu.SemaphoreType.DMA,
            pltpu.SemaphoreType.DMA,
        ],
    )(_sc_partial_body)
    return k(t2a, ids_a)


def _sc_final(t2b, ids_b, partial, bias):
    mesh = plsc.VectorSubcoreMesh(core_axis_name="c", subcore_axis_name="s")
    k = functools.partial(
        pl.kernel,
        out_type=jax.ShapeDtypeStruct((N, NOUT), jnp.float32),
        mesh=mesh,
        scratch_types=[
            pltpu.VMEM((_CHMAX * _IPCB,), jnp.int32),
            pltpu.VMEM((NOUT,), jnp.float32),
            pltpu.VMEM((_IPCB, _H), jnp.int32),
            pltpu.VMEM((_IPCB, _H), jnp.int32),
            pltpu.VMEM((_CHUNK, _H), jnp.int32),
            pltpu.VMEM((_CHUNK, _H), jnp.int32),
            pltpu.VMEM((_CHUNK, NOUT), jnp.float32),
            pltpu.SemaphoreType.DMA,
            pltpu.SemaphoreType.DMA,
            pltpu.SemaphoreType.DMA,
            pltpu.SemaphoreType.DMA,
        ],
    )(_sc_final_body)
    return k(t2b, ids_b, partial, bias)


def kernel(features, neighbor_idx, weight, bias):
    w = weight[:, 0, :, :].astype(jnp.bfloat16)
    f_bf = features.astype(jnp.bfloat16)
    ta = _matmul_offsets(f_bf, w[:_FVA])
    tb = _matmul_offsets(f_bf, w[_FVA:])
    idx32 = neighbor_idx.astype(jnp.int32)
    ids_a = idx32[:, :_FVA] + (jnp.arange(_FVA, dtype=jnp.int32) * N)[None, :]
    ids_a = jnp.pad(ids_a, ((0, NPAD - N), (0, 0))).reshape(-1)
    ids_b = idx32[:, _FVA:] + (jnp.arange(_FVB, dtype=jnp.int32) * N)[None, :]
    ids_b = jnp.pad(ids_b, ((0, NPAD - N), (0, 0))).reshape(-1)
    partial = _sc_partial(ta.reshape(_FVA * N, _H), ids_a)
    return _sc_final(tb.reshape(_FVB * N, _H), ids_b, partial,
                     bias.astype(jnp.float32))


# speedup vs baseline: 1.0271x; 1.0271x over previous
import functools

import jax
import jax.numpy as jnp
from jax import lax
from jax.experimental import pallas as pl
from jax.experimental.pallas import tpu as pltpu
from jax.experimental.pallas import tpu_sc as plsc

N = 10000
NIN = 512
NOUT = 512
FV = 13
_H = NOUT // 2

_QBITS = 12
_QSCALE = 512.0
_QOFF = 2048
_QMAX = 4095

_NC = 2
_NS = 16
_NW = _NC * _NS
_LANES = 16

NPAD = 10240
_CHUNK = 8
_IPC = _CHUNK * FV
_CH0 = 54
_CH1 = 26
_CHMAX = max(_CH0, _CH1)


def _quant(y):
    t = y * _QSCALE + (_QOFF + 0.5)
    t = jnp.clip(t, 0.0, float(_QMAX))
    return t.astype(jnp.int32)


def _mm_body(f_ref, w_ref, t_ref):
    y = jnp.dot(f_ref[...], w_ref[0], preferred_element_type=jnp.float32)
    qa = _quant(y[:, :_H])
    qb = _quant(y[:, _H:])
    t_ref[0] = qa | lax.shift_left(qb, 16)


def _matmul_all_offsets(features, w):
    blk = 2000
    grid = (N // blk, FV)
    return pl.pallas_call(
        _mm_body,
        grid=grid,
        in_specs=[
            pl.BlockSpec((blk, NIN), lambda nb, f: (nb, 0)),
            pl.BlockSpec((1, NIN, NOUT), lambda nb, f: (f, 0, 0)),
        ],
        out_specs=pl.BlockSpec((1, blk, _H), lambda nb, f: (f, nb, 0)),
        out_shape=jax.ShapeDtypeStruct((FV, N, _H), jnp.int32),
        compiler_params=pltpu.CompilerParams(
            dimension_semantics=("parallel", "parallel"),
        ),
    )(features, w)


_FVOFF = FV * _QOFF
_INV = 1.0 / _QSCALE


def _sc_body(t_hbm, ids_hbm, bias_hbm, out_hbm, ids_v, bias_v, g0, g1, outbuf,
             s0, s1):
    c = lax.axis_index("c")
    s = lax.axis_index("s")
    pltpu.sync_copy(bias_hbm, bias_v)

    def issue(ch, buf, sem):
        pltpu.async_copy(t_hbm.at[ids_v.at[pl.ds(ch * _IPC, _IPC)]], buf, sem)

    def wait(buf, sem):
        pltpu.make_async_copy(
            t_hbm.at[ids_v.at[pl.ds(0, _IPC)]], buf, sem
        ).wait()

    def consume(base_row, ch, buf):
        def grp(g, carry):
            sl = pl.ds(g * _LANES, _LANES)
            sh = pl.ds(_H + g * _LANES, _LANES)
            bl = bias_v[sl]
            bh = bias_v[sh]
            for r in range(_CHUNK):
                acc = buf[r * FV, sl]
                for f in range(1, FV):
                    acc = acc + buf[r * FV + f, sl]
                lo = (acc & 0xFFFF) - _FVOFF
                hi = lax.shift_right_logical(acc, 16) - _FVOFF
                outbuf[r, sl] = lo.astype(jnp.float32) * _INV + bl
                outbuf[r, sh] = hi.astype(jnp.float32) * _INV + bh
            return carry

        lax.fori_loop(0, _H // _LANES, grp, 0)

        @pl.when(base_row + ch * _CHUNK < N)
        def _():
            pltpu.sync_copy(
                outbuf, out_hbm.at[pl.ds(base_row + ch * _CHUNK, _CHUNK)]
            )

    def run(base_row, n_chunks):
        pltpu.sync_copy(
            ids_hbm.at[pl.ds(base_row * FV, n_chunks * _IPC)],
            ids_v.at[pl.ds(0, n_chunks * _IPC)],
        )
        issue(0, g0, s0)
        issue(1, g1, s1)

        def steady(i, carry):
            ch = 2 * i
            wait(g0, s0)
            consume(base_row, ch, g0)
            issue(ch + 2, g0, s0)
            wait(g1, s1)
            consume(base_row, ch + 1, g1)
            issue(ch + 3, g1, s1)
            return carry

        lax.fori_loop(0, n_chunks // 2 - 1, steady, 0)
        wait(g0, s0)
        consume(base_row, n_chunks - 2, g0)
        wait(g1, s1)
        consume(base_row, n_chunks - 1, g1)

    @pl.when(c == 0)
    def _():
        run(s * (_CH0 * _CHUNK), _CH0)

    @pl.when(c == 1)
    def _():
        run(_NS * _CH0 * _CHUNK + s * (_CH1 * _CHUNK), _CH1)


def _sc_gather_sum(t2, ids_flat, bias):
    mesh = plsc.VectorSubcoreMesh(core_axis_name="c", subcore_axis_name="s")
    k = functools.partial(
        pl.kernel,
        out_type=jax.ShapeDtypeStruct((N, NOUT), jnp.float32),
        mesh=mesh,
        scratch_types=[
            pltpu.VMEM((_CHMAX * _IPC,), jnp.int32),
            pltpu.VMEM((NOUT,), jnp.float32),
            pltpu.VMEM((_IPC, _H), jnp.int32),
            pltpu.VMEM((_IPC, _H), jnp.int32),
            pltpu.VMEM((_CHUNK, NOUT), jnp.float32),
            pltpu.SemaphoreType.DMA,
            pltpu.SemaphoreType.DMA,
        ],
    )(_sc_body)
    return k(t2, ids_flat, bias)


def kernel(features, neighbor_idx, weight, bias):
    w = weight[:, 0, :, :].astype(jnp.bfloat16)
    f_bf = features.astype(jnp.bfloat16)
    t = _matmul_all_offsets(f_bf, w)
    t2 = t.reshape(FV * N, _H)
    ids = neighbor_idx.astype(jnp.int32) + (
        jnp.arange(FV, dtype=jnp.int32) * N
    )[None, :]
    ids = jnp.pad(ids, ((0, NPAD - N), (0, 0))).reshape(-1)
    return _sc_gather_sum(t2, ids, bias.astype(jnp.float32))
